# Initial kernel scaffold; baseline (speedup 1.0000x reference)
import jax
import jax.numpy as jnp
from jax.experimental import pallas as pl


def kernel(xs, ys, read_in_w, read_in_b, wpe, read_out_w, read_out_b, ln_f_g, ln_f_b, b0_ln1_g, b0_ln1_b, b0_c_attn_w, b0_c_attn_b, b0_attn_proj_w, b0_attn_proj_b, b0_ln2_g, b0_ln2_b, b0_c_fc_w, b0_c_fc_b, b0_mlp_proj_w, b0_mlp_proj_b, b1_ln1_g, b1_ln1_b, b1_c_attn_w, b1_c_attn_b, b1_attn_proj_w, b1_attn_proj_b, b1_ln2_g, b1_ln2_b, b1_c_fc_w, b1_c_fc_b, b1_mlp_proj_w, b1_mlp_proj_b, b2_ln1_g, b2_ln1_b, b2_c_attn_w, b2_c_attn_b, b2_attn_proj_w, b2_attn_proj_b, b2_ln2_g, b2_ln2_b, b2_c_fc_w, b2_c_fc_b, b2_mlp_proj_w, b2_mlp_proj_b, b3_ln1_g, b3_ln1_b, b3_c_attn_w, b3_c_attn_b, b3_attn_proj_w, b3_attn_proj_b, b3_ln2_g, b3_ln2_b, b3_c_fc_w, b3_c_fc_b, b3_mlp_proj_w, b3_mlp_proj_b, b4_ln1_g, b4_ln1_b, b4_c_attn_w, b4_c_attn_b, b4_attn_proj_w, b4_attn_proj_b, b4_ln2_g, b4_ln2_b, b4_c_fc_w, b4_c_fc_b, b4_mlp_proj_w, b4_mlp_proj_b, b5_ln1_g, b5_ln1_b, b5_c_attn_w, b5_c_attn_b, b5_attn_proj_w, b5_attn_proj_b, b5_ln2_g, b5_ln2_b, b5_c_fc_w, b5_c_fc_b, b5_mlp_proj_w, b5_mlp_proj_b):
    raise NotImplementedError("write your pallas kernel here")


# speedup vs baseline: 3.4623x; 3.4623x over previous
import functools
import math

import jax
import jax.numpy as jnp
from jax.experimental import pallas as pl
from jax.experimental.pallas import tpu as pltpu

_BF16 = jnp.bfloat16
_F32 = jnp.float32
_VMEM_LIMIT = 64 * 1024 * 1024

_B, _P, _D, _E, _H = 64, 128, 64, 512, 8
_T = 2 * _P
_DH = _E // _H
_SCALE = 1.0 / math.sqrt(_DH)
_BB = 4
_RT = _BB * _T
_NB = _B // _BB
_LAYERS = 6
_CHUNKS = (6,)


def _ln(x, g, b):
    xf = x.astype(_F32)
    mu = jnp.mean(xf, axis=-1, keepdims=True)
    xc = xf - mu
    var = jnp.mean(xc * xc, axis=-1, keepdims=True)
    return (xc * jax.lax.rsqrt(var + 1e-5) * g + b).astype(_BF16)


def _read_in_kernel(z_ref, w_ref, b_ref, wpe_ref, o_ref):
    acc = jnp.dot(z_ref[...], w_ref[...], preferred_element_type=_F32)
    acc = acc + b_ref[...]
    acc = acc.reshape(_BB, _T, _E) + wpe_ref[...].astype(_F32)[None]
    o_ref[...] = acc.reshape(_RT, _E).astype(_BF16)


def _layers_kernel(*refs, n_layers):
    h_ref = refs[0]
    o_ref = refs[1 + 12 * n_layers]
    qkv_scr = refs[2 + 12 * n_layers]
    attn_scr = refs[3 + 12 * n_layers]

    row = jax.lax.broadcasted_iota(jnp.int32, (_T, _T), 0)
    col = jax.lax.broadcasted_iota(jnp.int32, (_T, _T), 1)
    causal = col <= row

    h = h_ref[...]
    for l in range(n_layers):
        (ln1_g, ln1_b, aw, ab, pw, pb,
         ln2_g, ln2_b, fw, fb, mw, mb) = refs[1 + 12 * l: 1 + 12 * (l + 1)]

        x = _ln(h, ln1_g[...], ln1_b[...])
        qkv = jnp.dot(x, aw[...], preferred_element_type=_F32) + ab[...]
        qkv_scr[...] = qkv.astype(_BF16)

        def _row(b, carry):
            base = b * _T
            qkvb = qkv_scr[pl.ds(base, _T), :]
            outs = []
            for hh in range(_H):
                qh = qkvb[:, hh * _DH:(hh + 1) * _DH] * _SCALE
                kh = qkvb[:, _E + hh * _DH:_E + (hh + 1) * _DH]
                vh = qkvb[:, 2 * _E + hh * _DH:2 * _E + (hh + 1) * _DH]
                s = jnp.dot(qh, kh.T, preferred_element_type=_F32)
                s = jnp.where(causal, s, -1e30)
                m = jnp.max(s, axis=-1, keepdims=True)
                p = jnp.exp(s - m)
                denom = jnp.sum(p, axis=-1, keepdims=True)
                p = p * pl.reciprocal(denom, approx=True)
                outs.append(jnp.dot(p.astype(_BF16), vh,
                                    preferred_element_type=_F32))
            attn_scr[pl.ds(base, _T), :] = jnp.concatenate(
                outs, axis=-1).astype(_BF16)
            return carry
        jax.lax.fori_loop(0, _BB, _row, 0)

        a = attn_scr[...]
        h = (jnp.dot(a, pw[...], preferred_element_type=_F32) + pb[...]
             + h.astype(_F32)).astype(_BF16)

        x2 = _ln(h, ln2_g[...], ln2_b[...])
        u = jnp.dot(x2, fw[...], preferred_element_type=_F32) + fb[...]
        c = 0.7978845608028654
        u = 0.5 * u * (1.0 + jnp.tanh(c * (u + 0.044715 * u * u * u)))
        mact = u.astype(_BF16)
        h = (jnp.dot(mact, mw[...], preferred_element_type=_F32) + mb[...]
             + h.astype(_F32)).astype(_BF16)

    o_ref[...] = h


def _readout_kernel(h_ref, g_ref, b_ref, w_ref, ob_ref, o_ref):
    x = _ln(h_ref[...], g_ref[...], b_ref[...])
    o_ref[...] = (jnp.dot(x, w_ref[...], preferred_element_type=_F32)
                  + ob_ref[...])


def _whole(arr):
    nd = arr.ndim
    return pl.BlockSpec(arr.shape, lambda i, _n=nd: (0,) * _n)


def _run_layers(h, layer_params):
    n = len(layer_params)
    in_specs = [pl.BlockSpec((_RT, _E), lambda i: (i, 0))]
    args = [h]
    for lp in layer_params:
        for arr in lp:
            in_specs.append(_whole(arr))
            args.append(arr)
    return pl.pallas_call(
        functools.partial(_layers_kernel, n_layers=n),
        grid=(_NB,),
        in_specs=in_specs,
        out_specs=pl.BlockSpec((_RT, _E), lambda i: (i, 0)),
        out_shape=jax.ShapeDtypeStruct((_B * _T, _E), _BF16),
        scratch_shapes=[pltpu.VMEM((_RT, 3 * _E), _BF16),
                        pltpu.VMEM((_RT, _E), _BF16)],
        compiler_params=pltpu.CompilerParams(
            dimension_semantics=("parallel",),
            vmem_limit_bytes=_VMEM_LIMIT,
        ),
    )(*args)


def kernel(xs, ys, read_in_w, read_in_b, wpe, read_out_w, read_out_b, ln_f_g, ln_f_b, b0_ln1_g, b0_ln1_b, b0_c_attn_w, b0_c_attn_b, b0_attn_proj_w, b0_attn_proj_b, b0_ln2_g, b0_ln2_b, b0_c_fc_w, b0_c_fc_b, b0_mlp_proj_w, b0_mlp_proj_b, b1_ln1_g, b1_ln1_b, b1_c_attn_w, b1_c_attn_b, b1_attn_proj_w, b1_attn_proj_b, b1_ln2_g, b1_ln2_b, b1_c_fc_w, b1_c_fc_b, b1_mlp_proj_w, b1_mlp_proj_b, b2_ln1_g, b2_ln1_b, b2_c_attn_w, b2_c_attn_b, b2_attn_proj_w, b2_attn_proj_b, b2_ln2_g, b2_ln2_b, b2_c_fc_w, b2_c_fc_b, b2_mlp_proj_w, b2_mlp_proj_b, b3_ln1_g, b3_ln1_b, b3_c_attn_w, b3_c_attn_b, b3_attn_proj_w, b3_attn_proj_b, b3_ln2_g, b3_ln2_b, b3_c_fc_w, b3_c_fc_b, b3_mlp_proj_w, b3_mlp_proj_b, b4_ln1_g, b4_ln1_b, b4_c_attn_w, b4_c_attn_b, b4_attn_proj_w, b4_attn_proj_b, b4_ln2_g, b4_ln2_b, b4_c_fc_w, b4_c_fc_b, b4_mlp_proj_w, b4_mlp_proj_b, b5_ln1_g, b5_ln1_b, b5_c_attn_w, b5_c_attn_b, b5_attn_proj_w, b5_attn_proj_b, b5_ln2_g, b5_ln2_b, b5_c_fc_w, b5_c_fc_b, b5_mlp_proj_w, b5_mlp_proj_b):
    blocks_flat = [
        b0_ln1_g, b0_ln1_b, b0_c_attn_w, b0_c_attn_b, b0_attn_proj_w, b0_attn_proj_b,
        b0_ln2_g, b0_ln2_b, b0_c_fc_w, b0_c_fc_b, b0_mlp_proj_w, b0_mlp_proj_b,
        b1_ln1_g, b1_ln1_b, b1_c_attn_w, b1_c_attn_b, b1_attn_proj_w, b1_attn_proj_b,
        b1_ln2_g, b1_ln2_b, b1_c_fc_w, b1_c_fc_b, b1_mlp_proj_w, b1_mlp_proj_b,
        b2_ln1_g, b2_ln1_b, b2_c_attn_w, b2_c_attn_b, b2_attn_proj_w, b2_attn_proj_b,
        b2_ln2_g, b2_ln2_b, b2_c_fc_w, b2_c_fc_b, b2_mlp_proj_w, b2_mlp_proj_b,
        b3_ln1_g, b3_ln1_b, b3_c_attn_w, b3_c_attn_b, b3_attn_proj_w, b3_attn_proj_b,
        b3_ln2_g, b3_ln2_b, b3_c_fc_w, b3_c_fc_b, b3_mlp_proj_w, b3_mlp_proj_b,
        b4_ln1_g, b4_ln1_b, b4_c_attn_w, b4_c_attn_b, b4_attn_proj_w, b4_attn_proj_b,
        b4_ln2_g, b4_ln2_b, b4_c_fc_w, b4_c_fc_b, b4_mlp_proj_w, b4_mlp_proj_b,
        b5_ln1_g, b5_ln1_b, b5_c_attn_w, b5_c_attn_b, b5_attn_proj_w, b5_attn_proj_b,
        b5_ln2_g, b5_ln2_b, b5_c_fc_w, b5_c_fc_b, b5_mlp_proj_w, b5_mlp_proj_b,
    ]
    layers = []
    for i in range(_LAYERS):
        (ln1_g, ln1_b, aw, ab, pw, pb,
         ln2_g, ln2_b, fw, fb, mw, mb) = blocks_flat[12 * i: 12 * (i + 1)]
        layers.append((
            ln1_g.reshape(1, _E).astype(_F32), ln1_b.reshape(1, _E).astype(_F32),
            aw, ab.reshape(1, 3 * _E).astype(_F32),
            pw, pb.reshape(1, _E).astype(_F32),
            ln2_g.reshape(1, _E).astype(_F32), ln2_b.reshape(1, _E).astype(_F32),
            fw, fb.reshape(1, 4 * _E).astype(_F32),
            mw, mb.reshape(1, _E).astype(_F32),
        ))

    ys_wide = jnp.concatenate(
        [ys[..., None], jnp.zeros((_B, _P, _D - 1), dtype=xs.dtype)], axis=2)
    zs = jnp.stack([xs, ys_wide], axis=2).reshape(_B, _T, _D).astype(_BF16)

    h = pl.pallas_call(
        _read_in_kernel,
        grid=(_NB,),
        in_specs=[
            pl.BlockSpec((_BB * _T, _D), lambda i: (i, 0)),
            _whole(read_in_w),
            pl.BlockSpec((1, _E), lambda i: (0, 0)),
            pl.BlockSpec((_T, _E), lambda i: (0, 0)),
        ],
        out_specs=pl.BlockSpec((_RT, _E), lambda i: (i, 0)),
        out_shape=jax.ShapeDtypeStruct((_B * _T, _E), _BF16),
        compiler_params=pltpu.CompilerParams(
            dimension_semantics=("parallel",),
            vmem_limit_bytes=_VMEM_LIMIT,
        ),
    )(zs.reshape(_B * _T, _D), read_in_w,
      read_in_b.reshape(1, _E).astype(_F32), wpe)

    off = 0
    for n in _CHUNKS:
        h = _run_layers(h, layers[off:off + n])
        off += n

    pred = pl.pallas_call(
        _readout_kernel,
        grid=(_NB,),
        in_specs=[
            pl.BlockSpec((_RT, _E), lambda i: (i, 0)),
            pl.BlockSpec((1, _E), lambda i: (0, 0)),
            pl.BlockSpec((1, _E), lambda i: (0, 0)),
            _whole(read_out_w),
            pl.BlockSpec((1, 1), lambda i: (0, 0)),
        ],
        out_specs=pl.BlockSpec((_RT, 1), lambda i: (i, 0)),
        out_shape=jax.ShapeDtypeStruct((_B * _T, 1), _F32),
        compiler_params=pltpu.CompilerParams(
            dimension_semantics=("parallel",),
            vmem_limit_bytes=_VMEM_LIMIT,
        ),
    )(h, ln_f_g.reshape(1, _E).astype(_F32), ln_f_b.reshape(1, _E).astype(_F32),
      read_out_w, read_out_b.reshape(1, 1).astype(_F32))

    return pred.reshape(_B, _T)[:, ::2]
